# baseline (speedup 1.0000x reference)
import jax
import jax.numpy as jnp
from jax import lax
from jax.experimental import pallas as pl
from jax.experimental.pallas import tpu as pltpu
from jax.experimental.pallas import tpu_sc as plsc

_BATCH = 4096
_FEAT = 4096
_NC = 2
_NS = 16
_L = 16
_NW = _NC * _NS
_ROWS_PER_W = _BATCH // _NW
_R = 8


def _shuffle_body(x_hbm, idx_hbm, out_hbm, idx_v, in_v, out_v):
    wid = lax.axis_index("s") * _NC + lax.axis_index("c")
    row0 = wid * _ROWS_PER_W

    pltpu.sync_copy(idx_hbm, idx_v)

    def chunk_body(ci, carry):
        base = (row0 + ci * _R) * _FEAT
        pltpu.sync_copy(x_hbm.at[pl.ds(base, _R * _FEAT)], in_v)

        def col_body(g, carry2):
            colv = idx_v[pl.ds(g * _L, _L)]
            for r in range(_R):
                vals = plsc.load_gather(in_v, [colv + (r * _FEAT)])
                out_v[pl.ds(r * _FEAT + g * _L, _L)] = vals
            return carry2

        lax.fori_loop(0, _FEAT // _L, col_body, 0, unroll=4)
        pltpu.sync_copy(out_v, out_hbm.at[pl.ds(base, _R * _FEAT)])
        return carry

    lax.fori_loop(0, _ROWS_PER_W // _R, chunk_body, 0)


@jax.jit
def _shuffle(x, indices):
    mesh = plsc.VectorSubcoreMesh(core_axis_name="c", subcore_axis_name="s")
    out_flat = pl.kernel(
        _shuffle_body,
        out_type=jax.ShapeDtypeStruct((_BATCH * _FEAT,), jnp.float32),
        mesh=mesh,
        scratch_types=[
            pltpu.VMEM((_FEAT,), jnp.int32),
            pltpu.VMEM((_R * _FEAT,), jnp.float32),
            pltpu.VMEM((_R * _FEAT,), jnp.float32),
        ],
        compiler_params=pltpu.CompilerParams(needs_layout_passes=False),
    )(x.reshape(-1), indices)
    return out_flat.reshape(_BATCH, _FEAT)


def kernel(x, indices):
    return _shuffle(x, indices)


# speedup vs baseline: 1.0488x; 1.0488x over previous
import jax
import jax.numpy as jnp
from jax import lax
from jax.experimental import pallas as pl
from jax.experimental.pallas import tpu as pltpu
from jax.experimental.pallas import tpu_sc as plsc

_BATCH = 4096
_FEAT = 4096
_NC = 2
_NS = 16
_L = 16
_NW = _NC * _NS
_ROWS_PER_W = _BATCH // _NW
_R = 4
_NCHUNKS = _ROWS_PER_W // _R
_CHUNK = _R * _FEAT


def _shuffle_body(x_hbm, idx_hbm, out_hbm, idx_v,
                  in_v0, in_v1, out_v0, out_v1,
                  in_sem0, in_sem1, out_sem0, out_sem1):
    wid = lax.axis_index("s") * _NC + lax.axis_index("c")
    row0 = wid * _ROWS_PER_W

    in_bufs = (in_v0, in_v1)
    out_bufs = (out_v0, out_v1)
    in_sems = (in_sem0, in_sem1)
    out_sems = (out_sem0, out_sem1)

    pltpu.sync_copy(idx_hbm, idx_v)

    def in_slice(ci):
        return x_hbm.at[pl.ds((row0 + ci * _R) * _FEAT, _CHUNK)]

    def out_slice(ci):
        return out_hbm.at[pl.ds((row0 + ci * _R) * _FEAT, _CHUNK)]

    for b in range(2):
        pltpu.async_copy(in_slice(b), in_bufs[b], in_sems[b])

    def outer(cj, carry):
        for b in range(2):
            ci = cj * 2 + b
            pltpu.make_async_copy(in_slice(ci), in_bufs[b], in_sems[b]).wait()

            @pl.when(cj > 0)
            def _wait_out():
                pltpu.make_async_copy(
                    out_bufs[b], out_slice(ci - 2), out_sems[b]
                ).wait()

            def col_body(g, carry2):
                colv = idx_v[pl.ds(g * _L, _L)]
                for r in range(_R):
                    vals = plsc.load_gather(in_bufs[b], [colv + (r * _FEAT)])
                    out_bufs[b][pl.ds(r * _FEAT + g * _L, _L)] = vals
                return carry2

            lax.fori_loop(0, _FEAT // _L, col_body, 0, unroll=4)

            pltpu.async_copy(out_bufs[b], out_slice(ci), out_sems[b])

            @pl.when(ci + 2 < _NCHUNKS)
            def _refill():
                pltpu.async_copy(in_slice(ci + 2), in_bufs[b], in_sems[b])

        return carry

    lax.fori_loop(0, _NCHUNKS // 2, outer, 0)

    for b in range(2):
        pltpu.make_async_copy(
            out_bufs[b], out_slice(_NCHUNKS - 2 + b), out_sems[b]
        ).wait()


@jax.jit
def _shuffle(x, indices):
    mesh = plsc.VectorSubcoreMesh(core_axis_name="c", subcore_axis_name="s")
    out_flat = pl.kernel(
        _shuffle_body,
        out_type=jax.ShapeDtypeStruct((_BATCH * _FEAT,), jnp.float32),
        mesh=mesh,
        scratch_types=[
            pltpu.VMEM((_FEAT,), jnp.int32),
            pltpu.VMEM((_CHUNK,), jnp.float32),
            pltpu.VMEM((_CHUNK,), jnp.float32),
            pltpu.VMEM((_CHUNK,), jnp.float32),
            pltpu.VMEM((_CHUNK,), jnp.float32),
            pltpu.SemaphoreType.DMA,
            pltpu.SemaphoreType.DMA,
            pltpu.SemaphoreType.DMA,
            pltpu.SemaphoreType.DMA,
        ],
        compiler_params=pltpu.CompilerParams(needs_layout_passes=False),
    )(x.reshape(-1), indices)
    return out_flat.reshape(_BATCH, _FEAT)


def kernel(x, indices):
    return _shuffle(x, indices)


# speedup vs baseline: 1.7740x; 1.6914x over previous
import jax
import jax.numpy as jnp
from jax import lax
from jax.experimental import pallas as pl
from jax.experimental.pallas import tpu as pltpu
from jax.experimental.pallas import tpu_sc as plsc

_BATCH = 4096
_FEAT = 4096
_NC = 2
_NS = 16
_L = 16
_NW = _NC * _NS
_ROWS_PER_W = _BATCH // _NW
_R = 4
_NCHUNKS = _ROWS_PER_W // _R
_CHUNK = _R * _FEAT


def _shuffle_body(x_hbm, idx_hbm, out_hbm, idx_v,
                  in_v0, in_v1, out_v0, out_v1,
                  in_sem0, in_sem1, out_sem0, out_sem1):
    wid = lax.axis_index("s") * _NC + lax.axis_index("c")
    row0 = wid * _ROWS_PER_W

    in_bufs = (in_v0, in_v1)
    out_bufs = (out_v0, out_v1)
    in_sems = (in_sem0, in_sem1)
    out_sems = (out_sem0, out_sem1)

    pltpu.sync_copy(idx_hbm, idx_v)

    def in_slice(ci):
        return x_hbm.at[pl.ds((row0 + ci * _R) * _FEAT, _CHUNK)]

    def out_slice(ci):
        return out_hbm.at[pl.ds((row0 + ci * _R) * _FEAT, _CHUNK)]

    for b in range(2):
        pltpu.async_copy(in_slice(b), in_bufs[b], in_sems[b])

    def outer(cj, carry):
        for b in range(2):
            ci = cj * 2 + b
            pltpu.make_async_copy(in_slice(ci), in_bufs[b], in_sems[b]).wait()

            @pl.when(cj > 0)
            def _wait_out():
                pltpu.make_async_copy(
                    out_bufs[b], out_slice(ci - 2), out_sems[b]
                ).wait()

            @plsc.parallel_loop(0, _FEAT // _L, unroll=4)
            def col_body(g):
                colv = idx_v[pl.ds(g * _L, _L)]
                for r in range(_R):
                    vals = plsc.load_gather(in_bufs[b], [colv + (r * _FEAT)])
                    out_bufs[b][pl.ds(r * _FEAT + g * _L, _L)] = vals

            pltpu.async_copy(out_bufs[b], out_slice(ci), out_sems[b])

            @pl.when(ci + 2 < _NCHUNKS)
            def _refill():
                pltpu.async_copy(in_slice(ci + 2), in_bufs[b], in_sems[b])

        return carry

    lax.fori_loop(0, _NCHUNKS // 2, outer, 0)

    for b in range(2):
        pltpu.make_async_copy(
            out_bufs[b], out_slice(_NCHUNKS - 2 + b), out_sems[b]
        ).wait()


@jax.jit
def _shuffle(x, indices):
    mesh = plsc.VectorSubcoreMesh(core_axis_name="c", subcore_axis_name="s")
    out_flat = pl.kernel(
        _shuffle_body,
        out_type=jax.ShapeDtypeStruct((_BATCH * _FEAT,), jnp.float32),
        mesh=mesh,
        scratch_types=[
            pltpu.VMEM((_FEAT,), jnp.int32),
            pltpu.VMEM((_CHUNK,), jnp.float32),
            pltpu.VMEM((_CHUNK,), jnp.float32),
            pltpu.VMEM((_CHUNK,), jnp.float32),
            pltpu.VMEM((_CHUNK,), jnp.float32),
            pltpu.SemaphoreType.DMA,
            pltpu.SemaphoreType.DMA,
            pltpu.SemaphoreType.DMA,
            pltpu.SemaphoreType.DMA,
        ],
        compiler_params=pltpu.CompilerParams(needs_layout_passes=False),
    )(x.reshape(-1), indices)
    return out_flat.reshape(_BATCH, _FEAT)


def kernel(x, indices):
    return _shuffle(x, indices)


# speedup vs baseline: 4.6851x; 2.6409x over previous
import jax
import jax.numpy as jnp
from jax import lax
from jax.experimental import pallas as pl
from jax.experimental.pallas import tpu as pltpu
from jax.experimental.pallas import tpu_sc as plsc

_BATCH = 4096
_FEAT = 4096
_NC = 2
_NS = 16
_L = 16
_NW = _NC * _NS
_ROWS_PER_W = _BATCH // _NW
_R = 4
_NCHUNKS = _ROWS_PER_W // _R
_CHUNK = _R * _FEAT


def _shuffle_body(x_hbm, idx_hbm, out_hbm, idx_v,
                  in_v0, in_v1, out_v0, out_v1,
                  in_sem0, in_sem1, out_sem0, out_sem1):
    wid = lax.axis_index("s") * _NC + lax.axis_index("c")
    row0 = wid * _ROWS_PER_W

    in_bufs = (in_v0, in_v1)
    out_bufs = (out_v0, out_v1)
    in_sems = (in_sem0, in_sem1)
    out_sems = (out_sem0, out_sem1)

    pltpu.sync_copy(idx_hbm, idx_v)

    def in_slice(ci):
        return x_hbm.at[pl.ds(row0 + ci * _R, _R), :]

    def out_slice(ci):
        return out_hbm.at[pl.ds(row0 + ci * _R, _R), :]

    for b in range(2):
        pltpu.async_copy(in_slice(b), in_bufs[b], in_sems[b])

    def outer(cj, carry):
        for b in range(2):
            ci = cj * 2 + b
            pltpu.make_async_copy(in_slice(ci), in_bufs[b], in_sems[b]).wait()

            @pl.when(cj > 0)
            def _wait_out():
                pltpu.make_async_copy(
                    out_bufs[b], out_slice(ci - 2), out_sems[b]
                ).wait()

            @plsc.parallel_loop(0, _FEAT // _L, unroll=4)
            def col_body(g):
                colv = idx_v[pl.ds(g * _L, _L)]
                for r in range(_R):
                    rowv = jnp.full((_L,), r, jnp.int32)
                    vals = plsc.load_gather(in_bufs[b], [rowv, colv])
                    out_bufs[b][r, pl.ds(g * _L, _L)] = vals

            pltpu.async_copy(out_bufs[b], out_slice(ci), out_sems[b])

            @pl.when(ci + 2 < _NCHUNKS)
            def _refill():
                pltpu.async_copy(in_slice(ci + 2), in_bufs[b], in_sems[b])

        return carry

    lax.fori_loop(0, _NCHUNKS // 2, outer, 0)

    for b in range(2):
        pltpu.make_async_copy(
            out_bufs[b], out_slice(_NCHUNKS - 2 + b), out_sems[b]
        ).wait()


@jax.jit
def _shuffle(x, indices):
    mesh = plsc.VectorSubcoreMesh(core_axis_name="c", subcore_axis_name="s")
    return pl.kernel(
        _shuffle_body,
        out_type=jax.ShapeDtypeStruct((_BATCH, _FEAT), jnp.float32),
        mesh=mesh,
        scratch_types=[
            pltpu.VMEM((_FEAT,), jnp.int32),
            pltpu.VMEM((_R, _FEAT), jnp.float32),
            pltpu.VMEM((_R, _FEAT), jnp.float32),
            pltpu.VMEM((_R, _FEAT), jnp.float32),
            pltpu.VMEM((_R, _FEAT), jnp.float32),
            pltpu.SemaphoreType.DMA,
            pltpu.SemaphoreType.DMA,
            pltpu.SemaphoreType.DMA,
            pltpu.SemaphoreType.DMA,
        ],
        compiler_params=pltpu.CompilerParams(needs_layout_passes=False),
    )(x, indices)


def kernel(x, indices):
    return _shuffle(x, indices)


# speedup vs baseline: 4.7131x; 1.0060x over previous
import jax
import jax.numpy as jnp
from jax import lax
from jax.experimental import pallas as pl
from jax.experimental.pallas import tpu as pltpu
from jax.experimental.pallas import tpu_sc as plsc

_BATCH = 4096
_FEAT = 4096
_NC = 2
_NS = 16
_L = 16
_NW = _NC * _NS
_ROWS_PER_W = _BATCH // _NW
_R = 4
_NCHUNKS = _ROWS_PER_W // _R
_CHUNK = _R * _FEAT


def _shuffle_body(x_hbm, idx_hbm, out_hbm, idx_v,
                  in_v0, in_v1, out_v0, out_v1,
                  in_sem0, in_sem1, out_sem0, out_sem1):
    wid = lax.axis_index("s") * _NC + lax.axis_index("c")
    row0 = wid * _ROWS_PER_W

    in_bufs = (in_v0, in_v1)
    out_bufs = (out_v0, out_v1)
    in_sems = (in_sem0, in_sem1)
    out_sems = (out_sem0, out_sem1)

    def in_slice(ci):
        return x_hbm.at[pl.ds(row0 + ci * _R, _R), :]

    def out_slice(ci):
        return out_hbm.at[pl.ds(row0 + ci * _R, _R), :]

    for b in range(2):
        pltpu.async_copy(in_slice(b), in_bufs[b], in_sems[b])

    pltpu.sync_copy(idx_hbm, idx_v)

    def outer(cj, carry):
        for b in range(2):
            ci = cj * 2 + b
            pltpu.make_async_copy(in_slice(ci), in_bufs[b], in_sems[b]).wait()

            @pl.when(cj > 0)
            def _wait_out():
                pltpu.make_async_copy(
                    out_bufs[b], out_slice(ci - 2), out_sems[b]
                ).wait()

            @plsc.parallel_loop(0, _FEAT // _L, unroll=4)
            def col_body(g):
                colv = idx_v[pl.ds(g * _L, _L)]
                for r in range(_R):
                    rowv = jnp.full((_L,), r, jnp.int32)
                    vals = plsc.load_gather(in_bufs[b], [rowv, colv])
                    out_bufs[b][r, pl.ds(g * _L, _L)] = vals

            pltpu.async_copy(out_bufs[b], out_slice(ci), out_sems[b])

            @pl.when(ci + 2 < _NCHUNKS)
            def _refill():
                pltpu.async_copy(in_slice(ci + 2), in_bufs[b], in_sems[b])

        return carry

    lax.fori_loop(0, _NCHUNKS // 2, outer, 0)

    for b in range(2):
        pltpu.make_async_copy(
            out_bufs[b], out_slice(_NCHUNKS - 2 + b), out_sems[b]
        ).wait()


@jax.jit
def _shuffle(x, indices):
    mesh = plsc.VectorSubcoreMesh(core_axis_name="c", subcore_axis_name="s")
    return pl.kernel(
        _shuffle_body,
        out_type=jax.ShapeDtypeStruct((_BATCH, _FEAT), jnp.float32),
        mesh=mesh,
        scratch_types=[
            pltpu.VMEM((_FEAT,), jnp.int32),
            pltpu.VMEM((_R, _FEAT), jnp.float32),
            pltpu.VMEM((_R, _FEAT), jnp.float32),
            pltpu.VMEM((_R, _FEAT), jnp.float32),
            pltpu.VMEM((_R, _FEAT), jnp.float32),
            pltpu.SemaphoreType.DMA,
            pltpu.SemaphoreType.DMA,
            pltpu.SemaphoreType.DMA,
            pltpu.SemaphoreType.DMA,
        ],
        compiler_params=pltpu.CompilerParams(needs_layout_passes=False),
    )(x, indices)


def kernel(x, indices):
    return _shuffle(x, indices)


# speedup vs baseline: 4.7343x; 1.0045x over previous
import jax
import jax.numpy as jnp
from jax import lax
from jax.experimental import pallas as pl
from jax.experimental.pallas import tpu as pltpu
from jax.experimental.pallas import tpu_sc as plsc

_BATCH = 4096
_FEAT = 4096
_NC = 2
_NS = 16
_L = 16
_NW = _NC * _NS
_ROWS_PER_W = _BATCH // _NW
_R = 4
_NCHUNKS = _ROWS_PER_W // _R
_CHUNK = _R * _FEAT


def _shuffle_body(x_hbm, idx_hbm, out_hbm, idx_v,
                  in_v0, in_v1, out_v0, out_v1,
                  in_sem0, in_sem1, out_sem0, out_sem1):
    wid = lax.axis_index("s") * _NC + lax.axis_index("c")
    row0 = wid * _ROWS_PER_W

    in_bufs = (in_v0, in_v1)
    out_bufs = (out_v0, out_v1)
    in_sems = (in_sem0, in_sem1)
    out_sems = (out_sem0, out_sem1)

    def in_slice(ci):
        return x_hbm.at[pl.ds(row0 + ci * _R, _R), :]

    def out_slice(ci):
        return out_hbm.at[pl.ds(row0 + ci * _R, _R), :]

    for b in range(2):
        pltpu.async_copy(in_slice(b), in_bufs[b], in_sems[b])

    pltpu.sync_copy(idx_hbm, idx_v)

    def outer(cj, carry):
        for b in range(2):
            ci = cj * 2 + b
            pltpu.make_async_copy(in_slice(ci), in_bufs[b], in_sems[b]).wait()

            @pl.when(cj > 0)
            def _wait_out():
                pltpu.make_async_copy(
                    out_bufs[b], out_slice(ci - 2), out_sems[b]
                ).wait()

            @plsc.parallel_loop(0, _FEAT // _L, unroll=4)
            def col_body(g):
                colv = idx_v[pl.ds(g * _L, _L)]
                for r in range(_R):
                    rowv = jnp.full((_L,), r, jnp.int32)
                    vals = plsc.load_gather(in_bufs[b], [rowv, colv])
                    out_bufs[b][r, pl.ds(g * _L, _L)] = vals

            pltpu.async_copy(out_bufs[b], out_slice(ci), out_sems[b])

            @pl.when(ci + 2 < _NCHUNKS)
            def _refill():
                pltpu.async_copy(in_slice(ci + 2), in_bufs[b], in_sems[b])

        return carry

    lax.fori_loop(0, _NCHUNKS // 2, outer, 0)

    for b in range(2):
        pltpu.make_async_copy(
            out_bufs[b], out_slice(_NCHUNKS - 2 + b), out_sems[b]
        ).wait()


@jax.jit
def _shuffle(x, indices):
    mesh = plsc.VectorSubcoreMesh(core_axis_name="c", subcore_axis_name="s")
    return pl.kernel(
        _shuffle_body,
        out_type=jax.ShapeDtypeStruct((_BATCH, _FEAT), jnp.float32),
        mesh=mesh,
        scratch_types=[
            pltpu.VMEM((_FEAT,), jnp.int32),
            pltpu.VMEM((_R, _FEAT), jnp.float32),
            pltpu.VMEM((_R, _FEAT), jnp.float32),
            pltpu.VMEM((_R, _FEAT), jnp.float32),
            pltpu.VMEM((_R, _FEAT), jnp.float32),
            pltpu.SemaphoreType.DMA,
            pltpu.SemaphoreType.DMA,
            pltpu.SemaphoreType.DMA,
            pltpu.SemaphoreType.DMA,
        ],
        compiler_params=pltpu.CompilerParams(
            needs_layout_passes=False, skip_device_barrier=True
        ),
    )(x, indices)


def kernel(x, indices):
    return _shuffle(x, indices)


# speedup vs baseline: 4.8869x; 1.0322x over previous
import jax
import jax.numpy as jnp
from jax import lax
from jax.experimental import pallas as pl
from jax.experimental.pallas import tpu as pltpu
from jax.experimental.pallas import tpu_sc as plsc

_BATCH = 4096
_FEAT = 4096
_NC = 2
_NS = 16
_L = 16
_NW = _NC * _NS
_ROWS_PER_W = _BATCH // _NW
_R = 2
_NBUF = 4
_NCHUNKS = _ROWS_PER_W // _R


def _shuffle_body(x_hbm, idx_hbm, out_hbm, idx_v, *bufs_and_sems):
    wid = lax.axis_index("s") * _NC + lax.axis_index("c")
    row0 = wid * _ROWS_PER_W

    in_bufs = bufs_and_sems[:_NBUF]
    out_bufs = bufs_and_sems[_NBUF:2 * _NBUF]
    in_sems = bufs_and_sems[2 * _NBUF:3 * _NBUF]
    out_sems = bufs_and_sems[3 * _NBUF:4 * _NBUF]

    def in_slice(ci):
        return x_hbm.at[pl.ds(row0 + ci * _R, _R), :]

    def out_slice(ci):
        return out_hbm.at[pl.ds(row0 + ci * _R, _R), :]

    for b in range(_NBUF):
        pltpu.async_copy(in_slice(b), in_bufs[b], in_sems[b])

    pltpu.sync_copy(idx_hbm, idx_v)

    def outer(cj, carry):
        for b in range(_NBUF):
            ci = cj * _NBUF + b
            pltpu.make_async_copy(in_slice(ci), in_bufs[b], in_sems[b]).wait()

            @pl.when(cj > 0)
            def _wait_out():
                pltpu.make_async_copy(
                    out_bufs[b], out_slice(ci - _NBUF), out_sems[b]
                ).wait()

            @plsc.parallel_loop(0, _FEAT // _L, unroll=4)
            def col_body(g):
                colv = idx_v[pl.ds(g * _L, _L)]
                for r in range(_R):
                    rowv = jnp.full((_L,), r, jnp.int32)
                    vals = plsc.load_gather(in_bufs[b], [rowv, colv])
                    out_bufs[b][r, pl.ds(g * _L, _L)] = vals

            pltpu.async_copy(out_bufs[b], out_slice(ci), out_sems[b])

            @pl.when(ci + _NBUF < _NCHUNKS)
            def _refill():
                pltpu.async_copy(in_slice(ci + _NBUF), in_bufs[b], in_sems[b])

        return carry

    lax.fori_loop(0, _NCHUNKS // _NBUF, outer, 0)

    for b in range(_NBUF):
        pltpu.make_async_copy(
            out_bufs[b], out_slice(_NCHUNKS - _NBUF + b), out_sems[b]
        ).wait()


@jax.jit
def _shuffle(x, indices):
    mesh = plsc.VectorSubcoreMesh(core_axis_name="c", subcore_axis_name="s")
    return pl.kernel(
        _shuffle_body,
        out_type=jax.ShapeDtypeStruct((_BATCH, _FEAT), jnp.float32),
        mesh=mesh,
        scratch_types=(
            [pltpu.VMEM((_FEAT,), jnp.int32)]
            + [pltpu.VMEM((_R, _FEAT), jnp.float32) for _ in range(2 * _NBUF)]
            + [pltpu.SemaphoreType.DMA for _ in range(2 * _NBUF)]
        ),
        compiler_params=pltpu.CompilerParams(
            needs_layout_passes=False, skip_device_barrier=True
        ),
    )(x, indices)


def kernel(x, indices):
    return _shuffle(x, indices)


# speedup vs baseline: 4.8986x; 1.0024x over previous
import jax
import jax.numpy as jnp
from jax import lax
from jax.experimental import pallas as pl
from jax.experimental.pallas import tpu as pltpu
from jax.experimental.pallas import tpu_sc as plsc

_BATCH = 4096
_FEAT = 4096
_NC = 2
_NS = 16
_L = 16
_NW = _NC * _NS
_ROWS_PER_W = _BATCH // _NW
_R = 2
_NIN = 8
_NOUT = 4
_NCHUNKS = _ROWS_PER_W // _R


def _shuffle_body(x_hbm, idx_hbm, out_hbm, idx_v, *bufs_and_sems):
    wid = lax.axis_index("s") * _NC + lax.axis_index("c")
    row0 = wid * _ROWS_PER_W

    in_bufs = bufs_and_sems[:_NIN]
    out_bufs = bufs_and_sems[_NIN:_NIN + _NOUT]
    in_sems = bufs_and_sems[_NIN + _NOUT:2 * _NIN + _NOUT]
    out_sems = bufs_and_sems[2 * _NIN + _NOUT:]

    def in_slice(ci):
        return x_hbm.at[pl.ds(row0 + ci * _R, _R), :]

    def out_slice(ci):
        return out_hbm.at[pl.ds(row0 + ci * _R, _R), :]

    for b in range(_NIN):
        pltpu.async_copy(in_slice(b), in_bufs[b], in_sems[b])

    pltpu.sync_copy(idx_hbm, idx_v)

    def outer(cj, carry):
        for b in range(_NIN):
            ci = cj * _NIN + b
            bo = b % _NOUT
            pltpu.make_async_copy(in_slice(ci), in_bufs[b], in_sems[b]).wait()

            @pl.when(ci >= _NOUT)
            def _wait_out():
                pltpu.make_async_copy(
                    out_bufs[bo], out_slice(ci - _NOUT), out_sems[bo]
                ).wait()

            @plsc.parallel_loop(0, _FEAT // _L, unroll=4)
            def col_body(g):
                colv = idx_v[pl.ds(g * _L, _L)]
                for r in range(_R):
                    rowv = jnp.full((_L,), r, jnp.int32)
                    vals = plsc.load_gather(in_bufs[b], [rowv, colv])
                    out_bufs[bo][r, pl.ds(g * _L, _L)] = vals

            pltpu.async_copy(out_bufs[bo], out_slice(ci), out_sems[bo])

            @pl.when(ci + _NIN < _NCHUNKS)
            def _refill():
                pltpu.async_copy(in_slice(ci + _NIN), in_bufs[b], in_sems[b])

        return carry

    lax.fori_loop(0, _NCHUNKS // _NIN, outer, 0)

    for b in range(_NOUT):
        ci = _NCHUNKS - _NOUT + b
        pltpu.make_async_copy(
            out_bufs[ci % _NOUT], out_slice(ci), out_sems[ci % _NOUT]
        ).wait()


@jax.jit
def _shuffle(x, indices):
    mesh = plsc.VectorSubcoreMesh(core_axis_name="c", subcore_axis_name="s")
    return pl.kernel(
        _shuffle_body,
        out_type=jax.ShapeDtypeStruct((_BATCH, _FEAT), jnp.float32),
        mesh=mesh,
        scratch_types=(
            [pltpu.VMEM((_FEAT,), jnp.int32)]
            + [pltpu.VMEM((_R, _FEAT), jnp.float32) for _ in range(_NIN + _NOUT)]
            + [pltpu.SemaphoreType.DMA for _ in range(_NIN + _NOUT)]
        ),
        compiler_params=pltpu.CompilerParams(
            needs_layout_passes=False, skip_device_barrier=True
        ),
    )(x, indices)


def kernel(x, indices):
    return _shuffle(x, indices)


# speedup vs baseline: 4.9243x; 1.0052x over previous
import jax
import jax.numpy as jnp
from jax import lax
from jax.experimental import pallas as pl
from jax.experimental.pallas import tpu as pltpu
from jax.experimental.pallas import tpu_sc as plsc

_BATCH = 4096
_FEAT = 4096
_NC = 2
_NS = 16
_L = 16
_NW = _NC * _NS
_ROWS_PER_W = _BATCH // _NW
_R = 4
_NIN = 4
_NOUT = 2
_NCHUNKS = _ROWS_PER_W // _R


def _shuffle_body(x_hbm, idx_hbm, out_hbm, idx_v, *bufs_and_sems):
    wid = lax.axis_index("s") * _NC + lax.axis_index("c")
    row0 = wid * _ROWS_PER_W

    in_bufs = bufs_and_sems[:_NIN]
    out_bufs = bufs_and_sems[_NIN:_NIN + _NOUT]
    in_sems = bufs_and_sems[_NIN + _NOUT:2 * _NIN + _NOUT]
    out_sems = bufs_and_sems[2 * _NIN + _NOUT:]

    def in_slice(ci):
        return x_hbm.at[pl.ds(row0 + ci * _R, _R), :]

    def out_slice(ci):
        return out_hbm.at[pl.ds(row0 + ci * _R, _R), :]

    for b in range(_NIN):
        pltpu.async_copy(in_slice(b), in_bufs[b], in_sems[b])

    pltpu.sync_copy(idx_hbm, idx_v)

    def outer(cj, carry):
        for b in range(_NIN):
            ci = cj * _NIN + b
            bo = b % _NOUT
            pltpu.make_async_copy(in_slice(ci), in_bufs[b], in_sems[b]).wait()

            @pl.when(ci >= _NOUT)
            def _wait_out():
                pltpu.make_async_copy(
                    out_bufs[bo], out_slice(ci - _NOUT), out_sems[bo]
                ).wait()

            @plsc.parallel_loop(0, _FEAT // _L, unroll=4)
            def col_body(g):
                colv = idx_v[pl.ds(g * _L, _L)]
                for r in range(_R):
                    rowv = jnp.full((_L,), r, jnp.int32)
                    vals = plsc.load_gather(in_bufs[b], [rowv, colv])
                    out_bufs[bo][r, pl.ds(g * _L, _L)] = vals

            pltpu.async_copy(out_bufs[bo], out_slice(ci), out_sems[bo])

            @pl.when(ci + _NIN < _NCHUNKS)
            def _refill():
                pltpu.async_copy(in_slice(ci + _NIN), in_bufs[b], in_sems[b])

        return carry

    lax.fori_loop(0, _NCHUNKS // _NIN, outer, 0)

    for b in range(_NOUT):
        ci = _NCHUNKS - _NOUT + b
        pltpu.make_async_copy(
            out_bufs[ci % _NOUT], out_slice(ci), out_sems[ci % _NOUT]
        ).wait()


@jax.jit
def _shuffle(x, indices):
    mesh = plsc.VectorSubcoreMesh(core_axis_name="c", subcore_axis_name="s")
    return pl.kernel(
        _shuffle_body,
        out_type=jax.ShapeDtypeStruct((_BATCH, _FEAT), jnp.float32),
        mesh=mesh,
        scratch_types=(
            [pltpu.VMEM((_FEAT,), jnp.int32)]
            + [pltpu.VMEM((_R, _FEAT), jnp.float32) for _ in range(_NIN + _NOUT)]
            + [pltpu.SemaphoreType.DMA for _ in range(_NIN + _NOUT)]
        ),
        compiler_params=pltpu.CompilerParams(
            needs_layout_passes=False, skip_device_barrier=True
        ),
    )(x, indices)


def kernel(x, indices):
    return _shuffle(x, indices)


# speedup vs baseline: 4.9604x; 1.0073x over previous
import jax
import jax.numpy as jnp
from jax import lax
from jax.experimental import pallas as pl
from jax.experimental.pallas import tpu as pltpu
from jax.experimental.pallas import tpu_sc as plsc

_BATCH = 4096
_FEAT = 4096
_NC = 2
_NS = 16
_L = 16
_NW = _NC * _NS
_ROWS_PER_W = _BATCH // _NW
_R = 4
_NIN = 4
_NOUT = 2
_NCHUNKS = _ROWS_PER_W // _R


def _shuffle_body(x_hbm, idx_hbm, out_hbm, idx_v, *bufs_and_sems):
    wid = lax.axis_index("s") * _NC + lax.axis_index("c")
    row0 = wid * _ROWS_PER_W

    in_bufs = bufs_and_sems[:_NIN]
    out_bufs = bufs_and_sems[_NIN:_NIN + _NOUT]
    in_sems = bufs_and_sems[_NIN + _NOUT:2 * _NIN + _NOUT]
    out_sems = bufs_and_sems[2 * _NIN + _NOUT:]

    def in_slice(ci):
        return x_hbm.at[pl.ds(row0 + ci * _R, _R), :]

    def out_slice(ci):
        return out_hbm.at[pl.ds(row0 + ci * _R, _R), :]

    for b in range(_NIN):
        pltpu.async_copy(in_slice(b), in_bufs[b], in_sems[b])

    pltpu.sync_copy(idx_hbm, idx_v)

    def outer(cj, carry):
        for b in range(_NIN):
            ci = cj * _NIN + b
            bo = b % _NOUT
            pltpu.make_async_copy(in_slice(ci), in_bufs[b], in_sems[b]).wait()

            @pl.when(ci >= _NOUT)
            def _wait_out():
                pltpu.make_async_copy(
                    out_bufs[bo], out_slice(ci - _NOUT), out_sems[bo]
                ).wait()

            @plsc.parallel_loop(0, _FEAT // _L, unroll=8)
            def col_body(g):
                colv = idx_v[pl.ds(g * _L, _L)]
                for r in range(_R):
                    rowv = jnp.full((_L,), r, jnp.int32)
                    vals = plsc.load_gather(in_bufs[b], [rowv, colv])
                    out_bufs[bo][r, pl.ds(g * _L, _L)] = vals

            pltpu.async_copy(out_bufs[bo], out_slice(ci), out_sems[bo])

            @pl.when(ci + _NIN < _NCHUNKS)
            def _refill():
                pltpu.async_copy(in_slice(ci + _NIN), in_bufs[b], in_sems[b])

        return carry

    lax.fori_loop(0, _NCHUNKS // _NIN, outer, 0)

    for b in range(_NOUT):
        ci = _NCHUNKS - _NOUT + b
        pltpu.make_async_copy(
            out_bufs[ci % _NOUT], out_slice(ci), out_sems[ci % _NOUT]
        ).wait()


@jax.jit
def _shuffle(x, indices):
    mesh = plsc.VectorSubcoreMesh(core_axis_name="c", subcore_axis_name="s")
    return pl.kernel(
        _shuffle_body,
        out_type=jax.ShapeDtypeStruct((_BATCH, _FEAT), jnp.float32),
        mesh=mesh,
        scratch_types=(
            [pltpu.VMEM((_FEAT,), jnp.int32)]
            + [pltpu.VMEM((_R, _FEAT), jnp.float32) for _ in range(_NIN + _NOUT)]
            + [pltpu.SemaphoreType.DMA for _ in range(_NIN + _NOUT)]
        ),
        compiler_params=pltpu.CompilerParams(
            needs_layout_passes=False, skip_device_barrier=True
        ),
    )(x, indices)


def kernel(x, indices):
    return _shuffle(x, indices)
